# baseline (speedup 1.0000x reference)
import functools

import jax
import jax.numpy as jnp
from jax import lax
from jax.experimental import pallas as pl
from jax.experimental.pallas import tpu as pltpu
from jax.experimental.pallas import tpu_sc as plsc

_THRESHOLD = 0.95
_ALPHA = 0.1
_N = 1000
_B = 16384
_LO = _ALPHA / _N
_HI = 1.0 - _ALPHA + _ALPHA / _N

_BLK = 2048
_G = _B // _BLK

_NC = 2
_NS = 16
_NW = _NC * _NS
_RPW = _B // _NW
_R = 128
_NCHUNK = _RPW // _R


def _stats_body(x_ref, t_ref, mask_ref, ps_ref):
    x = x_ref[...]
    m = jnp.max(x, axis=1, keepdims=True)
    s = jnp.sum(jnp.exp(x - m), axis=1, keepdims=True)
    gt = (1.0 / s) > _THRESHOLD
    cols = lax.broadcasted_iota(jnp.int32, (_BLK, _N), 1)
    amax = jnp.min(jnp.where(x == m, cols, _N), axis=1, keepdims=True)
    tgt = t_ref[0, 0, :].reshape(_BLK, 1)
    hot = jnp.where(gt, amax, tgt)
    mask_ref[0, 0, :] = gt.reshape(_BLK).astype(jnp.float32)
    ps_ref[0, 0, :] = hot.reshape(_BLK)


_stats = pl.pallas_call(
    _stats_body,
    grid=(_G,),
    in_specs=[
        pl.BlockSpec((_BLK, _N), lambda i: (i, 0)),
        pl.BlockSpec((1, 1, _BLK), lambda i: (i, 0, 0)),
    ],
    out_specs=[
        pl.BlockSpec((1, 1, _BLK), lambda i: (i, 0, 0)),
        pl.BlockSpec((1, 1, _BLK), lambda i: (i, 0, 0)),
    ],
    out_shape=[
        jax.ShapeDtypeStruct((_G, 1, _BLK), jnp.float32),
        jax.ShapeDtypeStruct((_G, 1, _BLK), jnp.int32),
    ],
)


@functools.partial(
    pl.kernel,
    out_type=jax.ShapeDtypeStruct((_B, _N), jnp.float32),
    mesh=plsc.VectorSubcoreMesh(core_axis_name="c", subcore_axis_name="s"),
    scratch_types=[
        pltpu.VMEM((_RPW,), jnp.int32),
        pltpu.VMEM((_R, _N), jnp.float32),
    ],
    compiler_params=pltpu.CompilerParams(
        use_tc_tiling_on_sc=False, needs_layout_passes=False
    ),
)
def _sc_write(ps_hbm, out_hbm, ps_v, buf):
    wid = lax.axis_index("s") * _NC + lax.axis_index("c")
    base = wid * _RPW
    pltpu.sync_copy(ps_hbm.at[pl.ds(base, _RPW)], ps_v)

    lo16 = jnp.full((16,), _LO, jnp.float32)
    hi16 = jnp.full((16,), _HI, jnp.float32)
    lane = lax.iota(jnp.int32, 16)

    def fill_row(r, _):
        def fill_col(c, carry):
            buf[r, pl.ds(c * 16, 16)] = lo16
            return carry
        lax.fori_loop(0, _N // 16, fill_col, _)
        buf[r, pl.ds(_N - 16, 16)] = lo16
        return _

    lax.fori_loop(0, _R, fill_row, 0)

    def do_chunk(k, _):
        def put(g, carry):
            p = ps_v[pl.ds(k * _R + g * 16, 16)]
            plsc.store_scatter(buf, [lane + g * 16, p], hi16)
            return carry
        lax.fori_loop(0, _R // 16, put, 0)
        pltpu.sync_copy(buf, out_hbm.at[pl.ds(base + k * _R, _R)])
        def unput(g, carry):
            p = ps_v[pl.ds(k * _R + g * 16, 16)]
            plsc.store_scatter(buf, [lane + g * 16, p], lo16)
            return carry
        return lax.fori_loop(0, _R // 16, unput, _)

    lax.fori_loop(0, _NCHUNK, do_chunk, 0)


def kernel(logits, targets):
    tg = targets.reshape(_G, 1, _BLK).astype(jnp.int32)
    mask3, ps3 = _stats(logits, tg)
    smooth = _sc_write(ps3.reshape(_B))
    return smooth, mask3.reshape(_B)


# speedup vs baseline: 6.2117x; 6.2117x over previous
import jax
import jax.numpy as jnp
from jax import lax
from jax.experimental import pallas as pl

_THRESHOLD = 0.95
_ALPHA = 0.1
_N = 1000
_B = 16384
_LO = _ALPHA / _N
_HI = 1.0 - _ALPHA + _ALPHA / _N

_BLKB = 2048
_G = _B // _BLKB


def _body(x_ref, t_ref, out_ref, mask_ref):
    x = x_ref[...]
    m = jnp.max(x, axis=0, keepdims=True)
    s = jnp.sum(jnp.exp(x - m), axis=0, keepdims=True)
    gt = (1.0 / s) > _THRESHOLD
    rows = lax.broadcasted_iota(jnp.int32, (_N, _BLKB), 0)
    amax = jnp.min(jnp.where(x == m, rows, _N), axis=0, keepdims=True)
    tgt = t_ref[0, 0, :].reshape(1, _BLKB)
    hot = jnp.where(gt, amax, tgt)
    out_ref[...] = jnp.where(rows == hot, _HI, _LO)
    mask_ref[...] = gt.reshape(_BLKB).astype(jnp.float32)


def kernel(logits, targets):
    xt = logits.T
    tg = targets.reshape(_G, 1, _BLKB).astype(jnp.int32)
    out_t, mask = pl.pallas_call(
        _body,
        grid=(_G,),
        in_specs=[
            pl.BlockSpec((_N, _BLKB), lambda i: (0, i)),
            pl.BlockSpec((1, 1, _BLKB), lambda i: (i, 0, 0)),
        ],
        out_specs=[
            pl.BlockSpec((_N, _BLKB), lambda i: (0, i)),
            pl.BlockSpec((_BLKB,), lambda i: (i,)),
        ],
        out_shape=[
            jax.ShapeDtypeStruct((_N, _B), jnp.float32),
            jax.ShapeDtypeStruct((_B,), jnp.float32),
        ],
    )(xt, tg)
    return out_t.T, mask
